# baseline (speedup 1.0000x reference)
import jax
import jax.numpy as jnp
from jax import lax
from jax.experimental import pallas as pl
from jax.experimental.pallas import tpu as pltpu
from jax.experimental.pallas import tpu_sc as plsc

_BATCH = 16384
_DIM = 32
_NUM_WORKERS = 32
_B_PER_W = _BATCH // _NUM_WORKERS


def _mf_body(user_ids_hbm, item_ids_hbm, user_emb_hbm, item_emb_hbm,
             out_hbm, uid_v, iid_v, urows_v, irows_v, out_v, sem_u, sem_i):
    num_cores = 2
    wid = lax.axis_index("s") * num_cores + lax.axis_index("c")
    base = wid * _B_PER_W

    pltpu.sync_copy(user_ids_hbm.at[pl.ds(base, _B_PER_W)], uid_v)
    pltpu.sync_copy(item_ids_hbm.at[pl.ds(base, _B_PER_W)], iid_v)

    cp_u = pltpu.async_copy(user_emb_hbm.at[uid_v], urows_v, sem_u)
    cp_i = pltpu.async_copy(item_emb_hbm.at[iid_v], irows_v, sem_i)
    cp_u.wait()
    cp_i.wait()

    lane = lax.iota(jnp.int32, 16)

    def group(g, _):
        row_idx = g * 16 + lane
        acc = jnp.zeros((16,), jnp.float32)
        for j in range(_DIM):
            col_idx = jnp.full((16,), j, jnp.int32)
            uu = plsc.load_gather(urows_v, [row_idx, col_idx])
            vv = plsc.load_gather(irows_v, [row_idx, col_idx])
            acc = acc + uu * vv
        out_v[pl.ds(g * 16, 16)] = acc
        return _

    lax.fori_loop(0, _B_PER_W // 16, group, None)

    pltpu.sync_copy(out_v, out_hbm.at[pl.ds(base, _B_PER_W)])


@jax.jit
def _mf(user_ids, item_ids, user_emb, item_emb):
    mesh = plsc.VectorSubcoreMesh(core_axis_name="c", subcore_axis_name="s")
    return pl.kernel(
        _mf_body,
        out_type=jax.ShapeDtypeStruct((_BATCH,), jnp.float32),
        mesh=mesh,
        scratch_types=[
            pltpu.VMEM((_B_PER_W,), jnp.int32),
            pltpu.VMEM((_B_PER_W,), jnp.int32),
            pltpu.VMEM((_B_PER_W, _DIM), jnp.float32),
            pltpu.VMEM((_B_PER_W, _DIM), jnp.float32),
            pltpu.VMEM((_B_PER_W,), jnp.float32),
            pltpu.SemaphoreType.DMA,
            pltpu.SemaphoreType.DMA,
        ],
        compiler_params=pltpu.CompilerParams(
            needs_layout_passes=False, use_tc_tiling_on_sc=False),
    )(user_ids, item_ids, user_emb, item_emb)


def kernel(user_ids, item_ids, user_emb, item_emb):
    return _mf(user_ids, item_ids, user_emb, item_emb)
